# baseline (speedup 1.0000x reference)
import functools

import jax
import jax.numpy as jnp
from jax import lax
from jax.experimental import pallas as pl

B, C, H, W = 2, 96, 256, 256
NUM_CLASS = 20
O = NUM_CLASS + 2


def _body(x_ref, wt_ref, b_ref, o_ref, *, tile: int):
    acc = jnp.dot(wt_ref[:], x_ref[0], preferred_element_type=jnp.float32)
    acc = acc + b_ref[:]
    row = lax.broadcasted_iota(jnp.int32, (O, tile), 0)
    o_ref[0] = jnp.where(row < NUM_CLASS, jax.nn.sigmoid(acc), acc)


@jax.jit
def kernel(x, W_hm, b_hm, W_wh, b_wh):
    hw = H * W
    tile = 32768
    xf = x.reshape(B, C, hw)
    wt = jnp.concatenate([W_hm, W_wh], axis=1).T
    b = jnp.concatenate([b_hm, b_wh])[:, None]

    out = pl.pallas_call(
        functools.partial(_body, tile=tile),
        grid=(B, hw // tile),
        in_specs=[
            pl.BlockSpec((1, C, tile), lambda b_, t: (b_, 0, t)),
            pl.BlockSpec((O, C), lambda b_, t: (0, 0)),
            pl.BlockSpec((O, 1), lambda b_, t: (0, 0)),
        ],
        out_specs=pl.BlockSpec((1, O, tile), lambda b_, t: (b_, 0, t)),
        out_shape=jax.ShapeDtypeStruct((B, O, hw), jnp.float32),
    )(xf, wt, b)
    return out.reshape(B, O, H, W)


# speedup vs baseline: 1.0067x; 1.0067x over previous
import functools

import jax
import jax.numpy as jnp
from jax import lax
from jax.experimental import pallas as pl

B, C, H, W = 2, 96, 256, 256
NUM_CLASS = 20
O = NUM_CLASS + 2


def _body(x_ref, wt_ref, b_ref, o_ref, *, tile: int):
    acc = jnp.dot(wt_ref[:], x_ref[0], preferred_element_type=jnp.float32)
    acc = acc + b_ref[:]
    row = lax.broadcasted_iota(jnp.int32, (O, tile), 0)
    o_ref[0] = jnp.where(row < NUM_CLASS, jax.nn.sigmoid(acc), acc)


@jax.jit
def kernel(x, W_hm, b_hm, W_wh, b_wh):
    hw = H * W
    tile = 16384
    xf = x.reshape(B, C, hw)
    wt = jnp.concatenate([W_hm, W_wh], axis=1).T
    b = jnp.concatenate([b_hm, b_wh])[:, None]

    out = pl.pallas_call(
        functools.partial(_body, tile=tile),
        grid=(B, hw // tile),
        in_specs=[
            pl.BlockSpec((1, C, tile), lambda b_, t: (b_, 0, t)),
            pl.BlockSpec((O, C), lambda b_, t: (0, 0)),
            pl.BlockSpec((O, 1), lambda b_, t: (0, 0)),
        ],
        out_specs=pl.BlockSpec((1, O, tile), lambda b_, t: (b_, 0, t)),
        out_shape=jax.ShapeDtypeStruct((B, O, hw), jnp.float32),
    )(xf, wt, b)
    return out.reshape(B, O, H, W)
